# baseline (speedup 1.0000x reference)
import functools

import jax
import jax.numpy as jnp
import numpy as np
from jax.experimental import pallas as pl
from jax.experimental.pallas import tpu as pltpu

N_POINTS = 8192
N1 = 1639
N2 = 410
NUM_FEATURES = 3
NUM_CLASSES = 40
MAX_NB = 64
BN_EPS = 1e-05


def _fps_body(n_samples, px_ref, py_ref, pz_ref, dinit_ref, iota_ref,
              idx_ref, coord_ref, dists_ref):
    lane = jax.lax.broadcasted_iota(jnp.int32, (1, 128), 1)
    iota = iota_ref[...]
    px = px_ref[...]
    py = py_ref[...]
    pz = pz_ref[...]
    dists_ref[...] = dinit_ref[...]
    idx_ref[...] = jnp.zeros(idx_ref.shape, jnp.int32)
    coord_ref[...] = jnp.zeros(coord_ref.shape, jnp.float32)

    def extract(sel_idx):
        m = iota == sel_idx
        sx = jnp.sum(jnp.where(m, px, 0.0))
        sy = jnp.sum(jnp.where(m, py, 0.0))
        sz = jnp.sum(jnp.where(m, pz, 0.0))
        return sx, sy, sz

    def store(i, sel_idx, sx, sy, sz):
        idx_ref[pl.ds(i, 1), :] = jnp.full((1, 128), sel_idx, jnp.int32)
        row = jnp.where(lane == 0, sx,
                        jnp.where(lane == 1, sy,
                                  jnp.where(lane == 2, sz, 0.0)))
        coord_ref[pl.ds(i, 1), :] = row.astype(jnp.float32)

    sx0, sy0, sz0 = extract(jnp.int32(0))
    store(0, jnp.int32(0), sx0, sy0, sz0)

    def body(i, carry):
        sx, sy, sz = carry
        dx = px - sx
        dy = py - sy
        dz = pz - sz
        d = dx * dx + dy * dy + dz * dz
        nd = jnp.minimum(dists_ref[...], d)
        dists_ref[...] = nd
        mval = jnp.max(nd)
        nxt = jnp.min(jnp.where(nd == mval, iota, jnp.int32(2**31 - 1)))
        s2 = extract(nxt)
        store(i, nxt, *s2)
        return s2

    jax.lax.fori_loop(1, n_samples, body, (sx0, sy0, sz0), unroll=False)


def _prep_planes(pos, npad):
    n = pos.shape[0]
    return jnp.pad(pos, ((0, npad - n), (0, 0))).T.reshape(3, npad // 128, 128)


def _run_fps(planes, n, n_samples, n_pad):
    r = planes.shape[1]
    npad = r * 128
    ar = jnp.arange(npad, dtype=jnp.int32).reshape(r, 128)
    dinit = jnp.where(ar < n, jnp.float32(1e30), jnp.float32(-1e30))
    idx_out, coord_out = pl.pallas_call(
        functools.partial(_fps_body, n_samples),
        out_shape=(jax.ShapeDtypeStruct((n_pad, 128), jnp.int32),
                   jax.ShapeDtypeStruct((n_pad, 128), jnp.float32)),
        scratch_shapes=[pltpu.VMEM((r, 128), jnp.float32)],
    )(planes[0], planes[1], planes[2], dinit, ar)
    return idx_out[:n_samples, 0], coord_out[:n_samples, :3], coord_out


_F32_INF_BITS = 0x7F800000


def _cumsum_lanes(x):
    p = x.shape[1]
    sh = 1
    while sh < p:
        x = x + jnp.concatenate(
            [jnp.zeros_like(x[:, :sh]), x[:, :-sh]], axis=1)
        sh *= 2
    return x


def _thresh_body(p_valid, rr, qc_ref, px_ref, py_ref, pz_ref, s_ref, n_ref):
    pdim = px_ref.shape[1]
    qx = qc_ref[:, 0:1]
    qy = qc_ref[:, 1:2]
    qz = qc_ref[:, 2:3]
    dx = qx - px_ref[...]
    dy = qy - py_ref[...]
    dz = qz - pz_ref[...]
    d2 = dx * dx + dy * dy + dz * dz
    jlane = jax.lax.broadcasted_iota(jnp.int32, (1, pdim), 1)
    inr = (d2 <= rr) & (jlane < p_valid)
    key = jnp.where(inr, jax.lax.bitcast_convert_type(d2, jnp.int32),
                    _F32_INF_BITS)
    v = jnp.zeros((qc_ref.shape[0], 1), jnp.int32)
    for b in range(30, -1, -1):
        cand = v | (1 << b)
        cnt = jnp.sum(jnp.where(key < cand, 1, 0), axis=1, keepdims=True)
        v = jnp.where(cnt >= MAX_NB, v, cand)
    cnt_less = jnp.sum(jnp.where(key < v, 1, 0), axis=1, keepdims=True)
    m = MAX_NB - cnt_less
    ties = (key == v) & (v < _F32_INF_BITS)
    tr = _cumsum_lanes(ties.astype(jnp.int32))
    sel = (key < v) | (ties & (tr <= m))
    rank = _cumsum_lanes(sel.astype(jnp.int32))
    s_ref[...] = jnp.where(sel, rank - 1, -1)
    n_ref[...] = jnp.broadcast_to(rank[:, pdim - 1:pdim], n_ref.shape)


def _run_thresh(q_rows, planes_flat, p_valid, rr, q_pad):
    pdim = planes_flat.shape[2]
    grid = (q_pad // 8,)
    pt_spec = pl.BlockSpec((1, pdim), lambda i: (0, 0))
    s_out, n_out = pl.pallas_call(
        functools.partial(_thresh_body, p_valid, rr),
        grid=grid,
        in_specs=[pl.BlockSpec((8, 128), lambda i: (i, 0)),
                  pt_spec, pt_spec, pt_spec],
        out_specs=[pl.BlockSpec((8, pdim), lambda i: (i, 0)),
                   pl.BlockSpec((8, 128), lambda i: (i, 0))],
        out_shape=(jax.ShapeDtypeStruct((q_pad, pdim), jnp.int32),
                   jax.ShapeDtypeStruct((q_pad, 128), jnp.int32)),
    )(q_rows, planes_flat[0], planes_flat[1], planes_flat[2])
    return s_out, n_out


_BN_INV = float(1.0 / np.sqrt(1.0 + BN_EPS))


def _final_body(x2_ref, c2_ref,
                w0x_ref, w0p_ref, b0_ref, g0_ref, t0_ref,
                w1_ref, b1_ref, g1_ref, t1_ref,
                w2_ref, b2_ref,
                h0_ref, hb0_ref, hg0_ref, ht0_ref,
                h1_ref, hb1_ref, hg1_ref, ht1_ref,
                h2_ref, hb2_ref,
                out_ref):
    x2 = x2_ref[...]
    cx = c2_ref[:, 0:1]
    cy = c2_ref[:, 1:2]
    cz = c2_ref[:, 2:3]
    y = (jnp.dot(x2, w0x_ref[...], preferred_element_type=jnp.float32)
         + cx * w0p_ref[0:1, :] + cy * w0p_ref[1:2, :] + cz * w0p_ref[2:3, :]
         + b0_ref[...])
    y = jax.nn.relu(y * (g0_ref[...] * _BN_INV) + t0_ref[...])
    y = jnp.dot(y, w1_ref[...], preferred_element_type=jnp.float32) + b1_ref[...]
    y = jax.nn.relu(y * (g1_ref[...] * _BN_INV) + t1_ref[...])
    h = jnp.dot(y, w2_ref[...], preferred_element_type=jnp.float32) + b2_ref[...]
    rows = jax.lax.broadcasted_iota(jnp.int32, h.shape, 0)
    h = jnp.where(rows < N2, h, -jnp.inf)
    g = jnp.max(h, axis=0, keepdims=True)
    g = jax.nn.relu((jnp.dot(g, h0_ref[...], preferred_element_type=jnp.float32)
                     + hb0_ref[...]) * (hg0_ref[...] * _BN_INV) + ht0_ref[...])
    g = jax.nn.relu((jnp.dot(g, h1_ref[...], preferred_element_type=jnp.float32)
                     + hb1_ref[...]) * (hg1_ref[...] * _BN_INV) + ht1_ref[...])
    logits = jnp.dot(g, h2_ref[...], preferred_element_type=jnp.float32) + hb2_ref[...]
    m = jnp.max(logits, axis=1, keepdims=True)
    s = jnp.log(jnp.sum(jnp.exp(logits - m), axis=1, keepdims=True))
    o = logits - m - s
    o = jnp.concatenate([o, jnp.zeros((1, 128 - NUM_CLASSES), jnp.float32)], axis=1)
    out_ref[...] = jnp.broadcast_to(o, (8, 128))


def _run_final(x2, ctr2_rows, p3, ph):
    x2p = jnp.pad(x2, ((0, 416 - N2), (0, 0)))
    (w0, b0, g0, t0), (w1, b1, g1, t1), (w2, b2) = p3
    (e0, f0, u0, v0), (e1, f1, u1, v1), (e2, f2) = ph
    row = lambda v: v.reshape(1, -1)
    res = pl.pallas_call(
        _final_body,
        out_shape=jax.ShapeDtypeStruct((8, 128), jnp.float32),
    )(x2p, ctr2_rows,
      w0[:512], w0[512:], row(b0), row(g0), row(t0),
      w1, row(b1), row(g1), row(t1),
      w2, row(b2),
      e0, row(f0), row(u0), row(v0),
      e1, row(f1), row(u1), row(v1),
      e2, row(f2))
    return jnp.broadcast_to(res[0:1, :NUM_CLASSES], (N_POINTS, NUM_CLASSES))


def _mlp_chain(layers, x):
    n = len(layers)
    for i, layer in enumerate(layers):
        x = x @ layer[0] + layer[1]
        if i < n - 1:
            x = x * (layer[2] / jnp.sqrt(1.0 + BN_EPS)) + layer[3]
            x = jax.nn.relu(x)
    return x


def _slotmap_to_nb(s_map, n_cnt):
    q, p = s_map.shape
    jr = jnp.broadcast_to(jnp.arange(p, dtype=jnp.int32)[None, :], (q, p))
    qi = jnp.broadcast_to(jnp.arange(q, dtype=jnp.int32)[:, None], (q, p))
    slot = jnp.where(s_map >= 0, s_map, MAX_NB)
    nb = jnp.zeros((q, MAX_NB + 1), jnp.int32).at[qi, slot].set(jr)[:, :MAX_NB]
    mask = jnp.arange(MAX_NB, dtype=jnp.int32)[None, :] < n_cnt[:, 0:1]
    return nb, mask


def kernel(x, batch, params):
    pos = x[:, :3]
    feat = x[:, 3:]

    planes1 = _prep_planes(pos, N_POINTS)
    idx1, ctr1, ctr1_rows = _run_fps(planes1, N_POINTS, N1, 1664)
    s1, ncnt1 = _run_thresh(ctr1_rows, planes1.reshape(3, 1, N_POINTS),
                            N_POINTS, 4.0, 1664)
    nb1_idx, nb1_mask = _slotmap_to_nb(s1[:N1], ncnt1[:N1])

    planes2 = _prep_planes(ctr1, 2048)
    idx2, ctr2, ctr2_rows = _run_fps(planes2, N1, N2, 416)
    s2, ncnt2 = _run_thresh(ctr2_rows, planes2.reshape(3, 1, 2048),
                            N1, 16.0, 416)
    nb2_idx, nb2_mask = _slotmap_to_nb(s2[:N2, :N1], ncnt2[:N2])

    msg1 = _mlp_chain(params['mlp1'],
                      jnp.concatenate([feat[nb1_idx],
                                       pos[nb1_idx] - ctr1[:, None, :]], axis=-1))
    msg1 = jnp.where(nb1_mask[:, :, None], msg1, -jnp.inf)
    x1 = jnp.max(msg1, axis=1)
    x1 = jnp.where(jnp.isfinite(x1), x1, 0.0)

    msg2 = _mlp_chain(params['mlp2'],
                      jnp.concatenate([x1[nb2_idx],
                                       ctr1[nb2_idx] - ctr2[:, None, :]], axis=-1))
    msg2 = jnp.where(nb2_mask[:, :, None], msg2, -jnp.inf)
    x2 = jnp.max(msg2, axis=1)
    x2 = jnp.where(jnp.isfinite(x2), x2, 0.0)

    return _run_final(x2, ctr2_rows, params['mlp3'], params['head'])


# speedup vs baseline: 22.5782x; 22.5782x over previous
import functools

import jax
import jax.numpy as jnp
import numpy as np
from jax import lax
from jax.experimental import pallas as pl
from jax.experimental.pallas import tpu as pltpu
from jax.experimental.pallas import tpu_sc as plsc

N_POINTS = 8192
N1 = 1639
N2 = 410
NUM_FEATURES = 3
NUM_CLASSES = 40
MAX_NB = 64
BN_EPS = 1e-05


def _fps_body(n_samples, px_ref, py_ref, pz_ref, dinit_ref, iota_ref,
              idx_ref, coord_ref, dists_ref):
    lane = jax.lax.broadcasted_iota(jnp.int32, (1, 128), 1)
    iota = iota_ref[...]
    px = px_ref[...]
    py = py_ref[...]
    pz = pz_ref[...]
    dists_ref[...] = dinit_ref[...]
    idx_ref[...] = jnp.zeros(idx_ref.shape, jnp.int32)
    coord_ref[...] = jnp.zeros(coord_ref.shape, jnp.float32)

    def extract(sel_idx):
        m = iota == sel_idx
        sx = jnp.sum(jnp.where(m, px, 0.0))
        sy = jnp.sum(jnp.where(m, py, 0.0))
        sz = jnp.sum(jnp.where(m, pz, 0.0))
        return sx, sy, sz

    def store(i, sel_idx, sx, sy, sz):
        idx_ref[pl.ds(i, 1), :] = jnp.full((1, 128), sel_idx, jnp.int32)
        row = jnp.where(lane == 0, sx,
                        jnp.where(lane == 1, sy,
                                  jnp.where(lane == 2, sz, 0.0)))
        coord_ref[pl.ds(i, 1), :] = row.astype(jnp.float32)

    sx0, sy0, sz0 = extract(jnp.int32(0))
    store(0, jnp.int32(0), sx0, sy0, sz0)

    def body(i, carry):
        sx, sy, sz = carry
        dx = px - sx
        dy = py - sy
        dz = pz - sz
        d = dx * dx + dy * dy + dz * dz
        nd = jnp.minimum(dists_ref[...], d)
        dists_ref[...] = nd
        mval = jnp.max(nd)
        nxt = jnp.min(jnp.where(nd == mval, iota, jnp.int32(2**31 - 1)))
        s2 = extract(nxt)
        store(i, nxt, *s2)
        return s2

    jax.lax.fori_loop(1, n_samples, body, (sx0, sy0, sz0), unroll=False)


def _prep_planes(pos, npad):
    n = pos.shape[0]
    return jnp.pad(pos, ((0, npad - n), (0, 0))).T.reshape(3, npad // 128, 128)


def _run_fps(planes, n, n_samples, n_pad):
    r = planes.shape[1]
    npad = r * 128
    ar = jnp.arange(npad, dtype=jnp.int32).reshape(r, 128)
    dinit = jnp.where(ar < n, jnp.float32(1e30), jnp.float32(-1e30))
    idx_out, coord_out = pl.pallas_call(
        functools.partial(_fps_body, n_samples),
        out_shape=(jax.ShapeDtypeStruct((n_pad, 128), jnp.int32),
                   jax.ShapeDtypeStruct((n_pad, 128), jnp.float32)),
        scratch_shapes=[pltpu.VMEM((r, 128), jnp.float32)],
    )(planes[0], planes[1], planes[2], dinit, ar)
    return idx_out[:n_samples, 0], coord_out[:n_samples, :3], coord_out


_F32_INF_BITS = 0x7F800000


def _cumsum_lanes(x):
    p = x.shape[1]
    sh = 1
    while sh < p:
        x = x + jnp.concatenate(
            [jnp.zeros_like(x[:, :sh]), x[:, :-sh]], axis=1)
        sh *= 2
    return x


def _thresh_body(p_valid, rr, qc_ref, px_ref, py_ref, pz_ref, s_ref, n_ref):
    pdim = px_ref.shape[1]
    qx = qc_ref[:, 0:1]
    qy = qc_ref[:, 1:2]
    qz = qc_ref[:, 2:3]
    dx = qx - px_ref[...]
    dy = qy - py_ref[...]
    dz = qz - pz_ref[...]
    d2 = dx * dx + dy * dy + dz * dz
    jlane = jax.lax.broadcasted_iota(jnp.int32, (1, pdim), 1)
    inr = (d2 <= rr) & (jlane < p_valid)
    key = jnp.where(inr, jax.lax.bitcast_convert_type(d2, jnp.int32),
                    _F32_INF_BITS)
    v = jnp.zeros((qc_ref.shape[0], 1), jnp.int32)
    for b in range(30, -1, -1):
        cand = v | (1 << b)
        cnt = jnp.sum(jnp.where(key < cand, 1, 0), axis=1, keepdims=True)
        v = jnp.where(cnt >= MAX_NB, v, cand)
    cnt_less = jnp.sum(jnp.where(key < v, 1, 0), axis=1, keepdims=True)
    m = MAX_NB - cnt_less
    ties = (key == v) & (v < _F32_INF_BITS)
    tr = _cumsum_lanes(ties.astype(jnp.int32))
    sel = (key < v) | (ties & (tr <= m))
    rank = _cumsum_lanes(sel.astype(jnp.int32))
    s_ref[...] = jnp.where(sel, rank - 1, -1)
    n_ref[...] = jnp.broadcast_to(rank[:, pdim - 1:pdim], n_ref.shape)


def _run_thresh(q_rows, planes_flat, p_valid, rr, q_pad):
    pdim = planes_flat.shape[2]
    grid = (q_pad // 8,)
    pt_spec = pl.BlockSpec((1, pdim), lambda i: (0, 0))
    s_out, n_out = pl.pallas_call(
        functools.partial(_thresh_body, p_valid, rr),
        grid=grid,
        in_specs=[pl.BlockSpec((8, 128), lambda i: (i, 0)),
                  pt_spec, pt_spec, pt_spec],
        out_specs=[pl.BlockSpec((8, pdim), lambda i: (i, 0)),
                   pl.BlockSpec((8, 128), lambda i: (i, 0))],
        out_shape=(jax.ShapeDtypeStruct((q_pad, pdim), jnp.int32),
                   jax.ShapeDtypeStruct((q_pad, 128), jnp.int32)),
    )(q_rows, planes_flat[0], planes_flat[1], planes_flat[2])
    return s_out, n_out


def _gather_rows(s_map, a_rows, q_pad, p_dim, d_dim):
    mesh = plsc.VectorSubcoreMesh(core_axis_name="c", subcore_axis_name="s")
    info = plsc.get_sparse_core_info()
    nw = info.num_cores * info.num_subcores
    qpw = q_pad // nw
    nchunk = p_dim // 16

    @functools.partial(
        pl.kernel, mesh=mesh,
        compiler_params=pltpu.CompilerParams(needs_layout_passes=False),
        out_type=jax.ShapeDtypeStruct((q_pad * MAX_NB, d_dim), jnp.float32),
        scratch_types=[
            pltpu.VMEM((p_dim,), jnp.int32),
            pltpu.VMEM((128,), jnp.int32),
            pltpu.VMEM((MAX_NB, d_dim), jnp.float32),
            pltpu.SemaphoreType.DMA,
        ],
    )
    def k(s_hbm, a_hbm, out_hbm, srow_v, nb_v, rows_v, sem):
        wid = lax.axis_index("s") * info.num_cores + lax.axis_index("c")
        base_q = wid * qpw

        def per_q(qi, _):
            q = base_q + qi
            pltpu.sync_copy(s_hbm.at[q], srow_v)
            for i in range(128 // 16):
                nb_v[pl.ds(i * 16, 16)] = jnp.zeros((16,), jnp.int32)

            def chunk(c, _):
                slots = srow_v[pl.ds(c * 16, 16)]
                jv = lax.iota(jnp.int32, 16) + c * 16
                plsc.store_scatter(nb_v, [slots], jv, mask=slots >= 0)
                return 0

            lax.fori_loop(0, nchunk, chunk, 0, unroll=4)
            pltpu.async_copy(a_hbm.at[nb_v.at[pl.ds(0, MAX_NB)]], rows_v,
                             sem).wait()
            pltpu.sync_copy(rows_v, out_hbm.at[pl.ds(q * MAX_NB, MAX_NB)])
            return 0

        lax.fori_loop(0, qpw, per_q, 0)

    return k(s_map, a_rows)


def _a1_body(xp_ref, w_ref, b_ref, qc_ref, wp_ref, a_ref, c_ref):
    a_ref[...] = (jnp.dot(xp_ref[...], w_ref[...],
                          preferred_element_type=jnp.float32) + b_ref[...])
    c_ref[...] = (qc_ref[:, 0:1] * wp_ref[0:1, :]
                  + qc_ref[:, 1:2] * wp_ref[1:2, :]
                  + qc_ref[:, 2:3] * wp_ref[2:3, :])


def _run_a1(xp, w_pad, b_row, q_rows, wp):
    return pl.pallas_call(
        _a1_body,
        out_shape=(jax.ShapeDtypeStruct((xp.shape[0], w_pad.shape[1]), jnp.float32),
                   jax.ShapeDtypeStruct((q_rows.shape[0], w_pad.shape[1]), jnp.float32)),
    )(xp, w_pad, b_row, q_rows, wp)


def _a2_body(x1_ref, w_ref, b_ref, pc_ref, wp_ref, qc_ref, a_ref, c_ref):
    a_ref[...] = (jnp.dot(x1_ref[...], w_ref[...],
                          preferred_element_type=jnp.float32)
                  + pc_ref[:, 0:1] * wp_ref[0:1, :]
                  + pc_ref[:, 1:2] * wp_ref[1:2, :]
                  + pc_ref[:, 2:3] * wp_ref[2:3, :]
                  + b_ref[...])
    c_ref[...] = (qc_ref[:, 0:1] * wp_ref[0:1, :]
                  + qc_ref[:, 1:2] * wp_ref[1:2, :]
                  + qc_ref[:, 2:3] * wp_ref[2:3, :])


def _run_a2(x1p, w, b_row, pc_rows, wp, q_rows):
    return pl.pallas_call(
        _a2_body,
        out_shape=(jax.ShapeDtypeStruct((x1p.shape[0], w.shape[1]), jnp.float32),
                   jax.ShapeDtypeStruct((q_rows.shape[0], w.shape[1]), jnp.float32)),
    )(x1p, w, b_row, pc_rows, wp, q_rows)


def _pair_body(g_ref, c_ref, n_ref,
               g1_ref, t1_ref, w2_ref, b2_ref, g2_ref, t2_ref,
               w3_ref, b3_ref, out_ref):
    qb, d_in = c_ref.shape
    y = g_ref[...]
    crep = jnp.broadcast_to(c_ref[...][:, None, :],
                            (qb, MAX_NB, d_in)).reshape(qb * MAX_NB, d_in)
    y = y - crep
    y = jax.nn.relu(y * (g1_ref[...] * _BN_INV) + t1_ref[...])
    y = jnp.dot(y, w2_ref[...], preferred_element_type=jnp.float32) + b2_ref[...]
    y = jax.nn.relu(y * (g2_ref[...] * _BN_INV) + t2_ref[...])
    y = jnp.dot(y, w3_ref[...], preferred_element_type=jnp.float32) + b3_ref[...]
    d_out = y.shape[1]
    y = y.reshape(qb, MAX_NB, d_out)
    slot = jax.lax.broadcasted_iota(jnp.int32, (qb, MAX_NB, 1), 1)
    y = jnp.where(slot < n_ref[:, 0:1].reshape(qb, 1, 1), y, -jnp.inf)
    m = jnp.max(y, axis=1)
    out_ref[...] = jnp.where(m == -jnp.inf, 0.0, m)


def _run_pair(g_rows, c_q, n_cnt, layers):
    (w1, b1, g1, t1), (w2, b2, g2, t2), (w3, b3) = layers
    q_pad, d_in = c_q.shape
    d_out = w3.shape[1]
    row = lambda v: v.reshape(1, -1)
    const = lambda shape: pl.BlockSpec(shape, lambda i: (0, 0))
    return pl.pallas_call(
        _pair_body,
        grid=(q_pad // 8,),
        in_specs=[pl.BlockSpec((8 * MAX_NB, d_in), lambda i: (i, 0)),
                  pl.BlockSpec((8, d_in), lambda i: (i, 0)),
                  pl.BlockSpec((8, 128), lambda i: (i, 0)),
                  const((1, d_in)), const((1, d_in)),
                  const(w2.shape), const((1, w2.shape[1])),
                  const((1, w2.shape[1])), const((1, w2.shape[1])),
                  const(w3.shape), const((1, d_out))],
        out_specs=pl.BlockSpec((8, d_out), lambda i: (i, 0)),
        out_shape=jax.ShapeDtypeStruct((q_pad, d_out), jnp.float32),
    )(g_rows, c_q, n_cnt,
      row(g1), row(t1), w2, row(b2), row(g2), row(t2), w3, row(b3))


_BN_INV = float(1.0 / np.sqrt(1.0 + BN_EPS))


def _final_body(x2_ref, c2_ref,
                w0x_ref, w0p_ref, b0_ref, g0_ref, t0_ref,
                w1_ref, b1_ref, g1_ref, t1_ref,
                w2_ref, b2_ref,
                h0_ref, hb0_ref, hg0_ref, ht0_ref,
                h1_ref, hb1_ref, hg1_ref, ht1_ref,
                h2_ref, hb2_ref,
                out_ref):
    x2 = x2_ref[...]
    cx = c2_ref[:, 0:1]
    cy = c2_ref[:, 1:2]
    cz = c2_ref[:, 2:3]
    y = (jnp.dot(x2, w0x_ref[...], preferred_element_type=jnp.float32)
         + cx * w0p_ref[0:1, :] + cy * w0p_ref[1:2, :] + cz * w0p_ref[2:3, :]
         + b0_ref[...])
    y = jax.nn.relu(y * (g0_ref[...] * _BN_INV) + t0_ref[...])
    y = jnp.dot(y, w1_ref[...], preferred_element_type=jnp.float32) + b1_ref[...]
    y = jax.nn.relu(y * (g1_ref[...] * _BN_INV) + t1_ref[...])
    h = jnp.dot(y, w2_ref[...], preferred_element_type=jnp.float32) + b2_ref[...]
    rows = jax.lax.broadcasted_iota(jnp.int32, h.shape, 0)
    h = jnp.where(rows < N2, h, -jnp.inf)
    g = jnp.max(h, axis=0, keepdims=True)
    g = jax.nn.relu((jnp.dot(g, h0_ref[...], preferred_element_type=jnp.float32)
                     + hb0_ref[...]) * (hg0_ref[...] * _BN_INV) + ht0_ref[...])
    g = jax.nn.relu((jnp.dot(g, h1_ref[...], preferred_element_type=jnp.float32)
                     + hb1_ref[...]) * (hg1_ref[...] * _BN_INV) + ht1_ref[...])
    logits = jnp.dot(g, h2_ref[...], preferred_element_type=jnp.float32) + hb2_ref[...]
    m = jnp.max(logits, axis=1, keepdims=True)
    s = jnp.log(jnp.sum(jnp.exp(logits - m), axis=1, keepdims=True))
    o = logits - m - s
    o = jnp.concatenate([o, jnp.zeros((1, 128 - NUM_CLASSES), jnp.float32)], axis=1)
    out_ref[...] = jnp.broadcast_to(o, (8, 128))


def _run_final(x2p, ctr2_rows, p3, ph):
    (w0, b0, g0, t0), (w1, b1, g1, t1), (w2, b2) = p3
    (e0, f0, u0, v0), (e1, f1, u1, v1), (e2, f2) = ph
    row = lambda v: v.reshape(1, -1)
    res = pl.pallas_call(
        _final_body,
        out_shape=jax.ShapeDtypeStruct((8, 128), jnp.float32),
    )(x2p, ctr2_rows,
      w0[:512], w0[512:], row(b0), row(g0), row(t0),
      w1, row(b1), row(g1), row(t1),
      w2, row(b2),
      e0, row(f0), row(u0), row(v0),
      e1, row(f1), row(u1), row(v1),
      e2, row(f2))
    return jnp.broadcast_to(res[0:1, :NUM_CLASSES], (N_POINTS, NUM_CLASSES))


def _mlp_chain(layers, x):
    n = len(layers)
    for i, layer in enumerate(layers):
        x = x @ layer[0] + layer[1]
        if i < n - 1:
            x = x * (layer[2] / jnp.sqrt(1.0 + BN_EPS)) + layer[3]
            x = jax.nn.relu(x)
    return x


def _slotmap_to_nb(s_map, n_cnt):
    q, p = s_map.shape
    jr = jnp.broadcast_to(jnp.arange(p, dtype=jnp.int32)[None, :], (q, p))
    qi = jnp.broadcast_to(jnp.arange(q, dtype=jnp.int32)[:, None], (q, p))
    slot = jnp.where(s_map >= 0, s_map, MAX_NB)
    nb = jnp.zeros((q, MAX_NB + 1), jnp.int32).at[qi, slot].set(jr)[:, :MAX_NB]
    mask = jnp.arange(MAX_NB, dtype=jnp.int32)[None, :] < n_cnt[:, 0:1]
    return nb, mask


def kernel(x, batch, params):
    pos = x[:, :3]
    feat = x[:, 3:]

    planes1 = _prep_planes(pos, N_POINTS)
    idx1, ctr1, ctr1_rows = _run_fps(planes1, N_POINTS, N1, 1664)
    s1, ncnt1 = _run_thresh(ctr1_rows, planes1.reshape(3, 1, N_POINTS),
                            N_POINTS, 4.0, 1664)

    planes2 = _prep_planes(ctr1, 2048)
    idx2, ctr2, ctr2_rows = _run_fps(planes2, N1, N2, 416)
    s2, ncnt2 = _run_thresh(ctr2_rows, planes2.reshape(3, 1, 2048),
                            N1, 16.0, 416)

    w1 = params['mlp1'][0][0]
    xp = jnp.concatenate([feat, pos, jnp.zeros((N_POINTS, 2), jnp.float32)],
                         axis=1)
    w1_pad = jnp.pad(w1, ((0, 2), (0, 0)))
    a1, c1 = _run_a1(xp, w1_pad, params['mlp1'][0][1].reshape(1, -1),
                     ctr1_rows, w1[3:6])
    g1_rows = _gather_rows(s1, a1, 1664, N_POINTS, 128)
    x1 = _run_pair(g1_rows, c1, ncnt1, params['mlp1'])

    w2 = params['mlp2'][0][0]
    x1p = jnp.pad(x1, ((0, 2048 - 1664), (0, 0)))
    pc_rows = jnp.pad(ctr1_rows, ((0, 2048 - 1664), (0, 0)))
    a2, c2 = _run_a2(x1p, w2[:256], params['mlp2'][0][1].reshape(1, -1),
                     pc_rows, w2[256:259], ctr2_rows)
    g2_rows = _gather_rows(s2, a2, 416, 2048, 256)
    x2 = _run_pair(g2_rows, c2, ncnt2, params['mlp2'])

    return _run_final(x2, ctr2_rows, params['mlp3'], params['head'])


# speedup vs baseline: 29.6062x; 1.3113x over previous
import functools

import jax
import jax.numpy as jnp
import numpy as np
from jax import lax
from jax.experimental import pallas as pl
from jax.experimental.pallas import tpu as pltpu
from jax.experimental.pallas import tpu_sc as plsc

N_POINTS = 8192
N1 = 1639
N2 = 410
NUM_FEATURES = 3
NUM_CLASSES = 40
MAX_NB = 64
BN_EPS = 1e-05


def _fps_body(n_samples, px_ref, py_ref, pz_ref, dinit_ref, iota_ref,
              idx_ref, coord_ref, dists_ref):
    lane = jax.lax.broadcasted_iota(jnp.int32, (1, 128), 1)
    iota = iota_ref[...]
    px = px_ref[...]
    py = py_ref[...]
    pz = pz_ref[...]
    dists_ref[...] = dinit_ref[...]
    idx_ref[...] = jnp.zeros(idx_ref.shape, jnp.int32)
    coord_ref[...] = jnp.zeros(coord_ref.shape, jnp.float32)

    def extract(sel_idx):
        m = iota == sel_idx
        sx = jnp.sum(jnp.where(m, px, 0.0))
        sy = jnp.sum(jnp.where(m, py, 0.0))
        sz = jnp.sum(jnp.where(m, pz, 0.0))
        return sx, sy, sz

    def store(i, sel_idx, sx, sy, sz):
        idx_ref[pl.ds(i, 1), :] = jnp.full((1, 128), sel_idx, jnp.int32)
        row = jnp.where(lane == 0, sx,
                        jnp.where(lane == 1, sy,
                                  jnp.where(lane == 2, sz, 0.0)))
        coord_ref[pl.ds(i, 1), :] = row.astype(jnp.float32)

    sx0, sy0, sz0 = extract(jnp.int32(0))
    store(0, jnp.int32(0), sx0, sy0, sz0)

    def body(i, carry):
        sx, sy, sz = carry
        dx = px - sx
        dy = py - sy
        dz = pz - sz
        d = dx * dx + dy * dy + dz * dz
        nd = jnp.minimum(dists_ref[...], d)
        dists_ref[...] = nd
        mval = jnp.max(nd)
        nxt = jnp.min(jnp.where(nd == mval, iota, jnp.int32(2**31 - 1)))
        s2 = extract(nxt)
        store(i, nxt, *s2)
        return s2

    jax.lax.fori_loop(1, n_samples, body, (sx0, sy0, sz0), unroll=False)


def _prep_planes(pos, npad):
    n = pos.shape[0]
    return jnp.pad(pos, ((0, npad - n), (0, 0))).T.reshape(3, npad // 128, 128)


def _run_fps(planes, n, n_samples, n_pad):
    r = planes.shape[1]
    npad = r * 128
    ar = jnp.arange(npad, dtype=jnp.int32).reshape(r, 128)
    dinit = jnp.where(ar < n, jnp.float32(1e30), jnp.float32(-1e30))
    idx_out, coord_out = pl.pallas_call(
        functools.partial(_fps_body, n_samples),
        out_shape=(jax.ShapeDtypeStruct((n_pad, 128), jnp.int32),
                   jax.ShapeDtypeStruct((n_pad, 128), jnp.float32)),
        scratch_shapes=[pltpu.VMEM((r, 128), jnp.float32)],
    )(planes[0], planes[1], planes[2], dinit, ar)
    return idx_out[:n_samples, 0], coord_out[:n_samples, :3], coord_out


_F32_INF_BITS = 0x7F800000


def _cumsum_lanes(x):
    p = x.shape[1]
    sh = 1
    while sh < p:
        x = x + jnp.concatenate(
            [jnp.zeros_like(x[:, :sh]), x[:, :-sh]], axis=1)
        sh *= 2
    return x


def _thresh_body(p_valid, rr, qc_ref, px_ref, py_ref, pz_ref, s_ref, n_ref):
    pdim = px_ref.shape[1]
    qx = qc_ref[:, 0:1]
    qy = qc_ref[:, 1:2]
    qz = qc_ref[:, 2:3]
    dx = qx - px_ref[...]
    dy = qy - py_ref[...]
    dz = qz - pz_ref[...]
    d2 = dx * dx + dy * dy + dz * dz
    jlane = jax.lax.broadcasted_iota(jnp.int32, (1, pdim), 1)
    inr = (d2 <= rr) & (jlane < p_valid)
    key = jnp.where(inr, jax.lax.bitcast_convert_type(d2, jnp.int32),
                    _F32_INF_BITS)
    v = jnp.zeros((qc_ref.shape[0], 1), jnp.int32)
    for b in range(30, -1, -1):
        cand = v | (1 << b)
        cnt = jnp.sum(jnp.where(key < cand, 1, 0), axis=1, keepdims=True)
        v = jnp.where(cnt >= MAX_NB, v, cand)
    cnt_less = jnp.sum(jnp.where(key < v, 1, 0), axis=1, keepdims=True)
    m = MAX_NB - cnt_less
    ties = (key == v) & (v < _F32_INF_BITS)
    tr = _cumsum_lanes(ties.astype(jnp.int32))
    sel = (key < v) | (ties & (tr <= m))
    rank = _cumsum_lanes(sel.astype(jnp.int32))
    s_ref[...] = jnp.where(sel, rank - 1, -1)
    n_ref[...] = jnp.broadcast_to(rank[:, pdim - 1:pdim], n_ref.shape)


def _run_thresh(q_rows, planes_flat, p_valid, rr, q_pad):
    pdim = planes_flat.shape[2]
    qb = 32
    grid = (q_pad // qb,)
    pt_spec = pl.BlockSpec((1, pdim), lambda i: (0, 0))
    s_out, n_out = pl.pallas_call(
        functools.partial(_thresh_body, p_valid, rr),
        grid=grid,
        in_specs=[pl.BlockSpec((qb, 128), lambda i: (i, 0)),
                  pt_spec, pt_spec, pt_spec],
        out_specs=[pl.BlockSpec((qb, pdim), lambda i: (i, 0)),
                   pl.BlockSpec((qb, 128), lambda i: (i, 0))],
        out_shape=(jax.ShapeDtypeStruct((q_pad, pdim), jnp.int32),
                   jax.ShapeDtypeStruct((q_pad, 128), jnp.int32)),
    )(q_rows, planes_flat[0], planes_flat[1], planes_flat[2])
    return s_out, n_out


def _gather_rows(s_map, a_rows, q_pad, p_dim, d_dim):
    mesh = plsc.VectorSubcoreMesh(core_axis_name="c", subcore_axis_name="s")
    info = plsc.get_sparse_core_info()
    nw = info.num_cores * info.num_subcores
    qpw = q_pad // nw
    nchunk = p_dim // 16

    @functools.partial(
        pl.kernel, mesh=mesh,
        compiler_params=pltpu.CompilerParams(needs_layout_passes=False),
        out_type=jax.ShapeDtypeStruct((q_pad * MAX_NB, d_dim), jnp.float32),
        scratch_types=[
            pltpu.VMEM((p_dim,), jnp.int32),
            pltpu.VMEM((128,), jnp.int32),
            pltpu.VMEM((MAX_NB, d_dim), jnp.float32),
            pltpu.SemaphoreType.DMA,
        ],
    )
    def k(s_hbm, a_hbm, out_hbm, srow_v, nb_v, rows_v, sem):
        wid = lax.axis_index("s") * info.num_cores + lax.axis_index("c")
        base_q = wid * qpw

        def per_q(qi, _):
            q = base_q + qi
            pltpu.sync_copy(s_hbm.at[q], srow_v)
            for i in range(128 // 16):
                nb_v[pl.ds(i * 16, 16)] = jnp.zeros((16,), jnp.int32)

            def chunk(c, _):
                slots = srow_v[pl.ds(c * 16, 16)]
                jv = lax.iota(jnp.int32, 16) + c * 16
                plsc.store_scatter(nb_v, [slots], jv, mask=slots >= 0)
                return 0

            lax.fori_loop(0, nchunk, chunk, 0, unroll=4)
            pltpu.async_copy(a_hbm.at[nb_v.at[pl.ds(0, MAX_NB)]], rows_v,
                             sem).wait()
            pltpu.sync_copy(rows_v, out_hbm.at[pl.ds(q * MAX_NB, MAX_NB)])
            return 0

        lax.fori_loop(0, qpw, per_q, 0)

    return k(s_map, a_rows)


def _a1_body(xp_ref, w_ref, b_ref, qc_ref, wp_ref, a_ref, c_ref):
    a_ref[...] = (jnp.dot(xp_ref[...], w_ref[...],
                          preferred_element_type=jnp.float32) + b_ref[...])
    c_ref[...] = (qc_ref[:, 0:1] * wp_ref[0:1, :]
                  + qc_ref[:, 1:2] * wp_ref[1:2, :]
                  + qc_ref[:, 2:3] * wp_ref[2:3, :])


def _run_a1(xp, w_pad, b_row, q_rows, wp):
    return pl.pallas_call(
        _a1_body,
        out_shape=(jax.ShapeDtypeStruct((xp.shape[0], w_pad.shape[1]), jnp.float32),
                   jax.ShapeDtypeStruct((q_rows.shape[0], w_pad.shape[1]), jnp.float32)),
    )(xp, w_pad, b_row, q_rows, wp)


def _a2_body(x1_ref, w_ref, b_ref, pc_ref, wp_ref, qc_ref, a_ref, c_ref):
    a_ref[...] = (jnp.dot(x1_ref[...], w_ref[...],
                          preferred_element_type=jnp.float32)
                  + pc_ref[:, 0:1] * wp_ref[0:1, :]
                  + pc_ref[:, 1:2] * wp_ref[1:2, :]
                  + pc_ref[:, 2:3] * wp_ref[2:3, :]
                  + b_ref[...])
    c_ref[...] = (qc_ref[:, 0:1] * wp_ref[0:1, :]
                  + qc_ref[:, 1:2] * wp_ref[1:2, :]
                  + qc_ref[:, 2:3] * wp_ref[2:3, :])


def _run_a2(x1p, w, b_row, pc_rows, wp, q_rows):
    return pl.pallas_call(
        _a2_body,
        out_shape=(jax.ShapeDtypeStruct((x1p.shape[0], w.shape[1]), jnp.float32),
                   jax.ShapeDtypeStruct((q_rows.shape[0], w.shape[1]), jnp.float32)),
    )(x1p, w, b_row, pc_rows, wp, q_rows)


def _pair_body(g_ref, c_ref, n_ref,
               g1_ref, t1_ref, w2_ref, b2_ref, g2_ref, t2_ref,
               w3_ref, b3_ref, out_ref):
    qb, d_in = c_ref.shape
    y = g_ref[...]
    crep = jnp.broadcast_to(c_ref[...][:, None, :],
                            (qb, MAX_NB, d_in)).reshape(qb * MAX_NB, d_in)
    y = y - crep
    y = jax.nn.relu(y * (g1_ref[...] * _BN_INV) + t1_ref[...])
    y = jnp.dot(y, w2_ref[...], preferred_element_type=jnp.float32) + b2_ref[...]
    y = jax.nn.relu(y * (g2_ref[...] * _BN_INV) + t2_ref[...])
    y = jnp.dot(y, w3_ref[...], preferred_element_type=jnp.float32) + b3_ref[...]
    d_out = y.shape[1]
    y = y.reshape(qb, MAX_NB, d_out)
    slot = jax.lax.broadcasted_iota(jnp.int32, (qb, MAX_NB, 1), 1)
    y = jnp.where(slot < n_ref[:, 0:1].reshape(qb, 1, 1), y, -jnp.inf)
    m = jnp.max(y, axis=1)
    out_ref[...] = jnp.where(m == -jnp.inf, 0.0, m)


def _run_pair(g_rows, c_q, n_cnt, layers):
    (w1, b1, g1, t1), (w2, b2, g2, t2), (w3, b3) = layers
    q_pad, d_in = c_q.shape
    d_out = w3.shape[1]
    row = lambda v: v.reshape(1, -1)
    const = lambda shape: pl.BlockSpec(shape, lambda i: (0, 0))
    return pl.pallas_call(
        _pair_body,
        grid=(q_pad // 8,),
        in_specs=[pl.BlockSpec((8 * MAX_NB, d_in), lambda i: (i, 0)),
                  pl.BlockSpec((8, d_in), lambda i: (i, 0)),
                  pl.BlockSpec((8, 128), lambda i: (i, 0)),
                  const((1, d_in)), const((1, d_in)),
                  const(w2.shape), const((1, w2.shape[1])),
                  const((1, w2.shape[1])), const((1, w2.shape[1])),
                  const(w3.shape), const((1, d_out))],
        out_specs=pl.BlockSpec((8, d_out), lambda i: (i, 0)),
        out_shape=jax.ShapeDtypeStruct((q_pad, d_out), jnp.float32),
    )(g_rows, c_q, n_cnt,
      row(g1), row(t1), w2, row(b2), row(g2), row(t2), w3, row(b3))


_BN_INV = float(1.0 / np.sqrt(1.0 + BN_EPS))


def _final_body(x2_ref, c2_ref,
                w0x_ref, w0p_ref, b0_ref, g0_ref, t0_ref,
                w1_ref, b1_ref, g1_ref, t1_ref,
                w2_ref, b2_ref,
                h0_ref, hb0_ref, hg0_ref, ht0_ref,
                h1_ref, hb1_ref, hg1_ref, ht1_ref,
                h2_ref, hb2_ref,
                out_ref):
    x2 = x2_ref[...]
    cx = c2_ref[:, 0:1]
    cy = c2_ref[:, 1:2]
    cz = c2_ref[:, 2:3]
    y = (jnp.dot(x2, w0x_ref[...], preferred_element_type=jnp.float32)
         + cx * w0p_ref[0:1, :] + cy * w0p_ref[1:2, :] + cz * w0p_ref[2:3, :]
         + b0_ref[...])
    y = jax.nn.relu(y * (g0_ref[...] * _BN_INV) + t0_ref[...])
    y = jnp.dot(y, w1_ref[...], preferred_element_type=jnp.float32) + b1_ref[...]
    y = jax.nn.relu(y * (g1_ref[...] * _BN_INV) + t1_ref[...])
    h = jnp.dot(y, w2_ref[...], preferred_element_type=jnp.float32) + b2_ref[...]
    rows = jax.lax.broadcasted_iota(jnp.int32, h.shape, 0)
    h = jnp.where(rows < N2, h, -jnp.inf)
    g = jnp.max(h, axis=0, keepdims=True)
    g = jax.nn.relu((jnp.dot(g, h0_ref[...], preferred_element_type=jnp.float32)
                     + hb0_ref[...]) * (hg0_ref[...] * _BN_INV) + ht0_ref[...])
    g = jax.nn.relu((jnp.dot(g, h1_ref[...], preferred_element_type=jnp.float32)
                     + hb1_ref[...]) * (hg1_ref[...] * _BN_INV) + ht1_ref[...])
    logits = jnp.dot(g, h2_ref[...], preferred_element_type=jnp.float32) + hb2_ref[...]
    m = jnp.max(logits, axis=1, keepdims=True)
    s = jnp.log(jnp.sum(jnp.exp(logits - m), axis=1, keepdims=True))
    o = logits - m - s
    o = jnp.concatenate([o, jnp.zeros((1, 128 - NUM_CLASSES), jnp.float32)], axis=1)
    out_ref[...] = jnp.broadcast_to(o, (8, 128))


def _run_final(x2p, ctr2_rows, p3, ph):
    (w0, b0, g0, t0), (w1, b1, g1, t1), (w2, b2) = p3
    (e0, f0, u0, v0), (e1, f1, u1, v1), (e2, f2) = ph
    row = lambda v: v.reshape(1, -1)
    res = pl.pallas_call(
        _final_body,
        out_shape=jax.ShapeDtypeStruct((8, 128), jnp.float32),
    )(x2p, ctr2_rows,
      w0[:512], w0[512:], row(b0), row(g0), row(t0),
      w1, row(b1), row(g1), row(t1),
      w2, row(b2),
      e0, row(f0), row(u0), row(v0),
      e1, row(f1), row(u1), row(v1),
      e2, row(f2))
    return jnp.broadcast_to(res[0:1, :NUM_CLASSES], (N_POINTS, NUM_CLASSES))


def _mlp_chain(layers, x):
    n = len(layers)
    for i, layer in enumerate(layers):
        x = x @ layer[0] + layer[1]
        if i < n - 1:
            x = x * (layer[2] / jnp.sqrt(1.0 + BN_EPS)) + layer[3]
            x = jax.nn.relu(x)
    return x


def _slotmap_to_nb(s_map, n_cnt):
    q, p = s_map.shape
    jr = jnp.broadcast_to(jnp.arange(p, dtype=jnp.int32)[None, :], (q, p))
    qi = jnp.broadcast_to(jnp.arange(q, dtype=jnp.int32)[:, None], (q, p))
    slot = jnp.where(s_map >= 0, s_map, MAX_NB)
    nb = jnp.zeros((q, MAX_NB + 1), jnp.int32).at[qi, slot].set(jr)[:, :MAX_NB]
    mask = jnp.arange(MAX_NB, dtype=jnp.int32)[None, :] < n_cnt[:, 0:1]
    return nb, mask


def kernel(x, batch, params):
    pos = x[:, :3]
    feat = x[:, 3:]

    planes1 = _prep_planes(pos, N_POINTS)
    idx1, ctr1, ctr1_rows = _run_fps(planes1, N_POINTS, N1, 1664)
    s1, ncnt1 = _run_thresh(ctr1_rows, planes1.reshape(3, 1, N_POINTS),
                            N_POINTS, 4.0, 1664)

    planes2 = _prep_planes(ctr1, 2048)
    idx2, ctr2, ctr2_rows = _run_fps(planes2, N1, N2, 416)
    s2, ncnt2 = _run_thresh(ctr2_rows, planes2.reshape(3, 1, 2048),
                            N1, 16.0, 416)

    w1 = params['mlp1'][0][0]
    xp = jnp.concatenate([feat, pos, jnp.zeros((N_POINTS, 2), jnp.float32)],
                         axis=1)
    w1_pad = jnp.pad(w1, ((0, 2), (0, 0)))
    a1, c1 = _run_a1(xp, w1_pad, params['mlp1'][0][1].reshape(1, -1),
                     ctr1_rows, w1[3:6])
    g1_rows = _gather_rows(s1, a1, 1664, N_POINTS, 128)
    x1 = _run_pair(g1_rows, c1, ncnt1, params['mlp1'])

    w2 = params['mlp2'][0][0]
    x1p = jnp.pad(x1, ((0, 2048 - 1664), (0, 0)))
    pc_rows = jnp.pad(ctr1_rows, ((0, 2048 - 1664), (0, 0)))
    a2, c2 = _run_a2(x1p, w2[:256], params['mlp2'][0][1].reshape(1, -1),
                     pc_rows, w2[256:259], ctr2_rows)
    g2_rows = _gather_rows(s2, a2, 416, 2048, 256)
    x2 = _run_pair(g2_rows, c2, ncnt2, params['mlp2'])

    return _run_final(x2, ctr2_rows, params['mlp3'], params['head'])


# speedup vs baseline: 30.3332x; 1.0246x over previous
import functools

import jax
import jax.numpy as jnp
import numpy as np
from jax import lax
from jax.experimental import pallas as pl
from jax.experimental.pallas import tpu as pltpu
from jax.experimental.pallas import tpu_sc as plsc

N_POINTS = 8192
N1 = 1639
N2 = 410
NUM_FEATURES = 3
NUM_CLASSES = 40
MAX_NB = 64
BN_EPS = 1e-05


def _fps_body(n_samples, px_ref, py_ref, pz_ref, rows_ref, dinit_ref,
              iota_ref, idx_ref, coord_ref, dists_ref):
    iota = iota_ref[...]
    px = px_ref[...]
    py = py_ref[...]
    pz = pz_ref[...]
    dists_ref[...] = dinit_ref[...]
    idx_ref[...] = jnp.zeros(idx_ref.shape, jnp.int32)
    coord_ref[...] = jnp.zeros(coord_ref.shape, jnp.float32)

    def extract(i, sel_idx):
        idx_ref[pl.ds(i, 1), :] = jnp.full((1, 128), sel_idx, jnp.int32)
        row = rows_ref[pl.ds(sel_idx, 1), pl.ds(0, 128)]
        coord_ref[pl.ds(i, 1), :] = row
        return row[0:1, 0:1], row[0:1, 1:2], row[0:1, 2:3]

    sx0, sy0, sz0 = extract(0, 0)

    def body(i, carry):
        sx, sy, sz = carry
        dx = px - sx
        dy = py - sy
        dz = pz - sz
        d = dx * dx + dy * dy + dz * dz
        nd = jnp.minimum(dists_ref[...], d)
        dists_ref[...] = nd
        mval = jnp.max(nd)
        nxt = jnp.min(jnp.where(nd == mval, iota, jnp.int32(2**31 - 1)))
        return extract(i, nxt)

    jax.lax.fori_loop(1, n_samples, body, (sx0, sy0, sz0), unroll=False)


def _prep_planes(pos, npad):
    n = pos.shape[0]
    return jnp.pad(pos, ((0, npad - n), (0, 0))).T.reshape(3, npad // 128, 128)


def _run_fps(planes, pos_rows, n, n_samples, n_pad):
    r = planes.shape[1]
    npad = r * 128
    ar = jnp.arange(npad, dtype=jnp.int32).reshape(r, 128)
    dinit = jnp.where(ar < n, jnp.float32(1e30), jnp.float32(-1e30))
    idx_out, coord_out = pl.pallas_call(
        functools.partial(_fps_body, n_samples),
        out_shape=(jax.ShapeDtypeStruct((n_pad, 128), jnp.int32),
                   jax.ShapeDtypeStruct((n_pad, 128), jnp.float32)),
        scratch_shapes=[pltpu.VMEM((r, 128), jnp.float32)],
    )(planes[0], planes[1], planes[2], pos_rows, dinit, ar)
    return idx_out[:n_samples, 0], coord_out[:n_samples, :3], coord_out


_F32_INF_BITS = 0x7F800000


def _cumsum_lanes(x):
    p = x.shape[1]
    sh = 1
    while sh < p:
        x = x + jnp.concatenate(
            [jnp.zeros_like(x[:, :sh]), x[:, :-sh]], axis=1)
        sh *= 2
    return x


def _thresh_body(p_valid, rr, qc_ref, px_ref, py_ref, pz_ref, s_ref, n_ref):
    pdim = px_ref.shape[1]
    qx = qc_ref[:, 0:1]
    qy = qc_ref[:, 1:2]
    qz = qc_ref[:, 2:3]
    dx = qx - px_ref[...]
    dy = qy - py_ref[...]
    dz = qz - pz_ref[...]
    d2 = dx * dx + dy * dy + dz * dz
    jlane = jax.lax.broadcasted_iota(jnp.int32, (1, pdim), 1)
    inr = (d2 <= rr) & (jlane < p_valid)
    key = jnp.where(inr, jax.lax.bitcast_convert_type(d2, jnp.int32),
                    _F32_INF_BITS)
    v = jnp.zeros((qc_ref.shape[0], 1), jnp.int32)
    for b in range(30, -1, -1):
        cand = v | (1 << b)
        cnt = jnp.sum(jnp.where(key < cand, 1, 0), axis=1, keepdims=True)
        v = jnp.where(cnt >= MAX_NB, v, cand)
    cnt_less = jnp.sum(jnp.where(key < v, 1, 0), axis=1, keepdims=True)
    m = MAX_NB - cnt_less
    ties = (key == v) & (v < _F32_INF_BITS)
    tr = _cumsum_lanes(ties.astype(jnp.int32))
    sel = (key < v) | (ties & (tr <= m))
    rank = _cumsum_lanes(sel.astype(jnp.int32))
    s_ref[...] = jnp.where(sel, rank - 1, -1)
    n_ref[...] = jnp.broadcast_to(rank[:, pdim - 1:pdim], n_ref.shape)


def _run_thresh(q_rows, planes_flat, p_valid, rr, q_pad):
    pdim = planes_flat.shape[2]
    qb = 32
    grid = (q_pad // qb,)
    pt_spec = pl.BlockSpec((1, pdim), lambda i: (0, 0))
    s_out, n_out = pl.pallas_call(
        functools.partial(_thresh_body, p_valid, rr),
        grid=grid,
        in_specs=[pl.BlockSpec((qb, 128), lambda i: (i, 0)),
                  pt_spec, pt_spec, pt_spec],
        out_specs=[pl.BlockSpec((qb, pdim), lambda i: (i, 0)),
                   pl.BlockSpec((qb, 128), lambda i: (i, 0))],
        out_shape=(jax.ShapeDtypeStruct((q_pad, pdim), jnp.int32),
                   jax.ShapeDtypeStruct((q_pad, 128), jnp.int32)),
    )(q_rows, planes_flat[0], planes_flat[1], planes_flat[2])
    return s_out, n_out


def _gather_rows(s_map, a_rows, q_pad, p_dim, d_dim):
    mesh = plsc.VectorSubcoreMesh(core_axis_name="c", subcore_axis_name="s")
    info = plsc.get_sparse_core_info()
    nw = info.num_cores * info.num_subcores
    qpw = q_pad // nw
    nchunk = p_dim // 16

    @functools.partial(
        pl.kernel, mesh=mesh,
        compiler_params=pltpu.CompilerParams(needs_layout_passes=False),
        out_type=jax.ShapeDtypeStruct((q_pad * MAX_NB, d_dim), jnp.float32),
        scratch_types=[
            pltpu.VMEM((p_dim,), jnp.int32),
            pltpu.VMEM((128,), jnp.int32),
            pltpu.VMEM((MAX_NB, d_dim), jnp.float32),
            pltpu.SemaphoreType.DMA,
        ],
    )
    def k(s_hbm, a_hbm, out_hbm, srow_v, nb_v, rows_v, sem):
        wid = lax.axis_index("s") * info.num_cores + lax.axis_index("c")
        base_q = wid * qpw

        def per_q(qi, _):
            q = base_q + qi
            pltpu.sync_copy(s_hbm.at[q], srow_v)
            for i in range(128 // 16):
                nb_v[pl.ds(i * 16, 16)] = jnp.zeros((16,), jnp.int32)

            def chunk(c, _):
                slots = srow_v[pl.ds(c * 16, 16)]
                jv = lax.iota(jnp.int32, 16) + c * 16
                plsc.store_scatter(nb_v, [slots], jv, mask=slots >= 0)
                return 0

            lax.fori_loop(0, nchunk, chunk, 0, unroll=4)
            pltpu.async_copy(a_hbm.at[nb_v.at[pl.ds(0, MAX_NB)]], rows_v,
                             sem).wait()
            pltpu.sync_copy(rows_v, out_hbm.at[pl.ds(q * MAX_NB, MAX_NB)])
            return 0

        lax.fori_loop(0, qpw, per_q, 0)

    return k(s_map, a_rows)


def _a1_body(xp_ref, w_ref, b_ref, qc_ref, wp_ref, a_ref, c_ref):
    a_ref[...] = (jnp.dot(xp_ref[...], w_ref[...],
                          preferred_element_type=jnp.float32) + b_ref[...])
    c_ref[...] = (qc_ref[:, 0:1] * wp_ref[0:1, :]
                  + qc_ref[:, 1:2] * wp_ref[1:2, :]
                  + qc_ref[:, 2:3] * wp_ref[2:3, :])


def _run_a1(xp, w_pad, b_row, q_rows, wp):
    return pl.pallas_call(
        _a1_body,
        out_shape=(jax.ShapeDtypeStruct((xp.shape[0], w_pad.shape[1]), jnp.float32),
                   jax.ShapeDtypeStruct((q_rows.shape[0], w_pad.shape[1]), jnp.float32)),
    )(xp, w_pad, b_row, q_rows, wp)


def _a2_body(x1_ref, w_ref, b_ref, pc_ref, wp_ref, qc_ref, a_ref, c_ref):
    a_ref[...] = (jnp.dot(x1_ref[...], w_ref[...],
                          preferred_element_type=jnp.float32)
                  + pc_ref[:, 0:1] * wp_ref[0:1, :]
                  + pc_ref[:, 1:2] * wp_ref[1:2, :]
                  + pc_ref[:, 2:3] * wp_ref[2:3, :]
                  + b_ref[...])
    c_ref[...] = (qc_ref[:, 0:1] * wp_ref[0:1, :]
                  + qc_ref[:, 1:2] * wp_ref[1:2, :]
                  + qc_ref[:, 2:3] * wp_ref[2:3, :])


def _run_a2(x1p, w, b_row, pc_rows, wp, q_rows):
    return pl.pallas_call(
        _a2_body,
        out_shape=(jax.ShapeDtypeStruct((x1p.shape[0], w.shape[1]), jnp.float32),
                   jax.ShapeDtypeStruct((q_rows.shape[0], w.shape[1]), jnp.float32)),
    )(x1p, w, b_row, pc_rows, wp, q_rows)


def _pair_body(g_ref, c_ref, n_ref,
               g1_ref, t1_ref, w2_ref, b2_ref, g2_ref, t2_ref,
               w3_ref, b3_ref, out_ref):
    qb, d_in = c_ref.shape
    y = g_ref[...]
    crep = jnp.broadcast_to(c_ref[...][:, None, :],
                            (qb, MAX_NB, d_in)).reshape(qb * MAX_NB, d_in)
    y = y - crep
    y = jax.nn.relu(y * (g1_ref[...] * _BN_INV) + t1_ref[...])
    y = jnp.dot(y, w2_ref[...], preferred_element_type=jnp.float32) + b2_ref[...]
    y = jax.nn.relu(y * (g2_ref[...] * _BN_INV) + t2_ref[...])
    y = jnp.dot(y, w3_ref[...], preferred_element_type=jnp.float32) + b3_ref[...]
    d_out = y.shape[1]
    y = y.reshape(qb, MAX_NB, d_out)
    slot = jax.lax.broadcasted_iota(jnp.int32, (qb, MAX_NB, 1), 1)
    y = jnp.where(slot < n_ref[:, 0:1].reshape(qb, 1, 1), y, -jnp.inf)
    m = jnp.max(y, axis=1)
    out_ref[...] = jnp.where(m == -jnp.inf, 0.0, m)


def _run_pair(g_rows, c_q, n_cnt, layers):
    (w1, b1, g1, t1), (w2, b2, g2, t2), (w3, b3) = layers
    q_pad, d_in = c_q.shape
    d_out = w3.shape[1]
    row = lambda v: v.reshape(1, -1)
    const = lambda shape: pl.BlockSpec(shape, lambda i: (0, 0))
    return pl.pallas_call(
        _pair_body,
        grid=(q_pad // 8,),
        in_specs=[pl.BlockSpec((8 * MAX_NB, d_in), lambda i: (i, 0)),
                  pl.BlockSpec((8, d_in), lambda i: (i, 0)),
                  pl.BlockSpec((8, 128), lambda i: (i, 0)),
                  const((1, d_in)), const((1, d_in)),
                  const(w2.shape), const((1, w2.shape[1])),
                  const((1, w2.shape[1])), const((1, w2.shape[1])),
                  const(w3.shape), const((1, d_out))],
        out_specs=pl.BlockSpec((8, d_out), lambda i: (i, 0)),
        out_shape=jax.ShapeDtypeStruct((q_pad, d_out), jnp.float32),
    )(g_rows, c_q, n_cnt,
      row(g1), row(t1), w2, row(b2), row(g2), row(t2), w3, row(b3))


_BN_INV = float(1.0 / np.sqrt(1.0 + BN_EPS))


def _final_body(x2_ref, c2_ref,
                w0x_ref, w0p_ref, b0_ref, g0_ref, t0_ref,
                w1_ref, b1_ref, g1_ref, t1_ref,
                w2_ref, b2_ref,
                h0_ref, hb0_ref, hg0_ref, ht0_ref,
                h1_ref, hb1_ref, hg1_ref, ht1_ref,
                h2_ref, hb2_ref,
                out_ref):
    x2 = x2_ref[...]
    cx = c2_ref[:, 0:1]
    cy = c2_ref[:, 1:2]
    cz = c2_ref[:, 2:3]
    y = (jnp.dot(x2, w0x_ref[...], preferred_element_type=jnp.float32)
         + cx * w0p_ref[0:1, :] + cy * w0p_ref[1:2, :] + cz * w0p_ref[2:3, :]
         + b0_ref[...])
    y = jax.nn.relu(y * (g0_ref[...] * _BN_INV) + t0_ref[...])
    y = jnp.dot(y, w1_ref[...], preferred_element_type=jnp.float32) + b1_ref[...]
    y = jax.nn.relu(y * (g1_ref[...] * _BN_INV) + t1_ref[...])
    h = jnp.dot(y, w2_ref[...], preferred_element_type=jnp.float32) + b2_ref[...]
    rows = jax.lax.broadcasted_iota(jnp.int32, h.shape, 0)
    h = jnp.where(rows < N2, h, -jnp.inf)
    g = jnp.max(h, axis=0, keepdims=True)
    g = jax.nn.relu((jnp.dot(g, h0_ref[...], preferred_element_type=jnp.float32)
                     + hb0_ref[...]) * (hg0_ref[...] * _BN_INV) + ht0_ref[...])
    g = jax.nn.relu((jnp.dot(g, h1_ref[...], preferred_element_type=jnp.float32)
                     + hb1_ref[...]) * (hg1_ref[...] * _BN_INV) + ht1_ref[...])
    logits = jnp.dot(g, h2_ref[...], preferred_element_type=jnp.float32) + hb2_ref[...]
    m = jnp.max(logits, axis=1, keepdims=True)
    s = jnp.log(jnp.sum(jnp.exp(logits - m), axis=1, keepdims=True))
    o = logits - m - s
    o = jnp.concatenate([o, jnp.zeros((1, 128 - NUM_CLASSES), jnp.float32)], axis=1)
    out_ref[...] = jnp.broadcast_to(o, (8, 128))


def _run_final(x2p, ctr2_rows, p3, ph):
    (w0, b0, g0, t0), (w1, b1, g1, t1), (w2, b2) = p3
    (e0, f0, u0, v0), (e1, f1, u1, v1), (e2, f2) = ph
    row = lambda v: v.reshape(1, -1)
    res = pl.pallas_call(
        _final_body,
        out_shape=jax.ShapeDtypeStruct((8, 128), jnp.float32),
    )(x2p, ctr2_rows,
      w0[:512], w0[512:], row(b0), row(g0), row(t0),
      w1, row(b1), row(g1), row(t1),
      w2, row(b2),
      e0, row(f0), row(u0), row(v0),
      e1, row(f1), row(u1), row(v1),
      e2, row(f2))
    return jnp.broadcast_to(res[0:1, :NUM_CLASSES], (N_POINTS, NUM_CLASSES))


def _mlp_chain(layers, x):
    n = len(layers)
    for i, layer in enumerate(layers):
        x = x @ layer[0] + layer[1]
        if i < n - 1:
            x = x * (layer[2] / jnp.sqrt(1.0 + BN_EPS)) + layer[3]
            x = jax.nn.relu(x)
    return x


def _slotmap_to_nb(s_map, n_cnt):
    q, p = s_map.shape
    jr = jnp.broadcast_to(jnp.arange(p, dtype=jnp.int32)[None, :], (q, p))
    qi = jnp.broadcast_to(jnp.arange(q, dtype=jnp.int32)[:, None], (q, p))
    slot = jnp.where(s_map >= 0, s_map, MAX_NB)
    nb = jnp.zeros((q, MAX_NB + 1), jnp.int32).at[qi, slot].set(jr)[:, :MAX_NB]
    mask = jnp.arange(MAX_NB, dtype=jnp.int32)[None, :] < n_cnt[:, 0:1]
    return nb, mask


def kernel(x, batch, params):
    pos = x[:, :3]
    feat = x[:, 3:]

    planes1 = _prep_planes(pos, N_POINTS)
    pos_rows1 = jnp.pad(pos, ((0, 0), (0, 125)))
    idx1, ctr1, ctr1_rows = _run_fps(planes1, pos_rows1, N_POINTS, N1, 1664)
    s1, ncnt1 = _run_thresh(ctr1_rows, planes1.reshape(3, 1, N_POINTS),
                            N_POINTS, 4.0, 1664)

    planes2 = _prep_planes(ctr1, 2048)
    pos_rows2 = jnp.pad(ctr1_rows, ((0, 2048 - 1664), (0, 0)))
    idx2, ctr2, ctr2_rows = _run_fps(planes2, pos_rows2, N1, N2, 416)
    s2, ncnt2 = _run_thresh(ctr2_rows, planes2.reshape(3, 1, 2048),
                            N1, 16.0, 416)

    w1 = params['mlp1'][0][0]
    xp = jnp.concatenate([feat, pos, jnp.zeros((N_POINTS, 2), jnp.float32)],
                         axis=1)
    w1_pad = jnp.pad(w1, ((0, 2), (0, 0)))
    a1, c1 = _run_a1(xp, w1_pad, params['mlp1'][0][1].reshape(1, -1),
                     ctr1_rows, w1[3:6])
    g1_rows = _gather_rows(s1, a1, 1664, N_POINTS, 128)
    x1 = _run_pair(g1_rows, c1, ncnt1, params['mlp1'])

    w2 = params['mlp2'][0][0]
    x1p = jnp.pad(x1, ((0, 2048 - 1664), (0, 0)))
    pc_rows = jnp.pad(ctr1_rows, ((0, 2048 - 1664), (0, 0)))
    a2, c2 = _run_a2(x1p, w2[:256], params['mlp2'][0][1].reshape(1, -1),
                     pc_rows, w2[256:259], ctr2_rows)
    g2_rows = _gather_rows(s2, a2, 416, 2048, 256)
    x2 = _run_pair(g2_rows, c2, ncnt2, params['mlp2'])

    return _run_final(x2, ctr2_rows, params['mlp3'], params['head'])
